# Initial kernel scaffold; baseline (speedup 1.0000x reference)
import jax
import jax.numpy as jnp
from jax.experimental import pallas as pl


def kernel(node_features, edge_features, senders, receivers, params):
    raise NotImplementedError("write your pallas kernel here")


# speedup vs baseline: 2.0311x; 2.0311x over previous
import functools

import jax
import jax.numpy as jnp
from jax import lax
from jax.experimental import pallas as pl
from jax.experimental.pallas import tpu as pltpu
from jax.experimental.pallas import tpu_sc as plsc

N = 10000
E = 320000
D = 128
EP = 327680
NPAD = 10240
NC = 2
NS = 16
PER_TILE = EP // (NC * NS)
GCH = 128
GITERS = PER_TILE // GCH

BLK_E = 2048
BLK_N = 2000

_F32 = jnp.float32


def _dot(a, b):
    return jnp.dot(a, b, preferred_element_type=_F32)


def _ln(x, g, b):
    m = jnp.mean(x, axis=-1, keepdims=True)
    xc = x - m
    v = jnp.mean(xc * xc, axis=-1, keepdims=True)
    return xc * lax.rsqrt(v + 1e-5) * g + b


def _full(shape):
    nd = len(shape)
    return pl.BlockSpec(shape, lambda i: (0,) * nd)


def _mlp3_ln_body(x, w1, b1, w2, b2, w3, b3, g, b, out):
    h = jnp.maximum(_dot(x[...], w1[...]) + b1[...], 0.0)
    h = jnp.maximum(_dot(h, w2[...]) + b2[...], 0.0)
    h = _dot(h, w3[...]) + b3[...]
    out[...] = _ln(h, g[...], b[...])


def _mlp3_ln(x, mlp, lng, lnb, blk):
    (w1, b1), (w2, b2), (w3, b3) = mlp
    n, din = x.shape
    return pl.pallas_call(
        _mlp3_ln_body,
        grid=(n // blk,),
        in_specs=[
            pl.BlockSpec((blk, din), lambda i: (i, 0)),
            _full((din, D)), _full((1, D)),
            _full((D, D)), _full((1, D)),
            _full((D, D)), _full((1, D)),
            _full((1, D)), _full((1, D)),
        ],
        out_specs=pl.BlockSpec((blk, D), lambda i: (i, 0)),
        out_shape=jax.ShapeDtypeStruct((n, D), _F32),
    )(x, w1, b1.reshape(1, D), w2, b2.reshape(1, D), w3, b3.reshape(1, D),
      lng.reshape(1, D), lnb.reshape(1, D))


def _proj2_body(nl, ws, wr, ps_out, pr_out):
    x = nl[...]
    ps_out[...] = _dot(x, ws[...])
    pr_out[...] = _dot(x, wr[...])


def _proj2(node_lat, ws, wr):
    return pl.pallas_call(
        _proj2_body,
        grid=(N // BLK_N,),
        in_specs=[
            pl.BlockSpec((BLK_N, D), lambda i: (i, 0)),
            _full((D, D)), _full((D, D)),
        ],
        out_specs=[pl.BlockSpec((BLK_N, D), lambda i: (i, 0))] * 2,
        out_shape=[jax.ShapeDtypeStruct((N, D), _F32)] * 2,
    )(node_lat, ws, wr)


def _edge_step_body(gs, gr, el, w1e, b1, w2, b2, w3, b3, g, b, ne_out, el_out):
    x = gs[...] + gr[...] + _dot(el[...], w1e[...]) + b1[...]
    x = jnp.maximum(x, 0.0)
    x = jnp.maximum(_dot(x, w2[...]) + b2[...], 0.0)
    x = _dot(x, w3[...]) + b3[...]
    ne = _ln(x, g[...], b[...])
    ne_out[...] = ne
    el_out[...] = el[...] + ne


def _edge_step(gs, gr, el, w1e, b1, w2, b2, w3, b3, lng, lnb):
    row = pl.BlockSpec((BLK_E, D), lambda i: (i, 0))
    return pl.pallas_call(
        _edge_step_body,
        grid=(EP // BLK_E,),
        in_specs=[row, row, row,
                  _full((D, D)), _full((1, D)),
                  _full((D, D)), _full((1, D)),
                  _full((D, D)), _full((1, D)),
                  _full((1, D)), _full((1, D))],
        out_specs=[row, row],
        out_shape=[jax.ShapeDtypeStruct((EP, D), _F32)] * 2,
    )(gs, gr, el, w1e, b1.reshape(1, D), w2, b2.reshape(1, D),
      w3, b3.reshape(1, D), lng.reshape(1, D), lnb.reshape(1, D))


def _node_step_body(nl, a0, a1, wna, wnb, b1, w2, b2, w3, b3, g, b, out):
    nlv = nl[...]
    agg = a0[...] + a1[...]
    x = _dot(nlv, wna[...]) + _dot(agg, wnb[...]) + b1[...]
    x = jnp.maximum(x, 0.0)
    x = jnp.maximum(_dot(x, w2[...]) + b2[...], 0.0)
    x = _dot(x, w3[...]) + b3[...]
    out[...] = nlv + _ln(x, g[...], b[...])


def _node_step(node_lat, a0, a1, wna, wnb, b1, w2, b2, w3, b3, lng, lnb):
    row = pl.BlockSpec((BLK_N, D), lambda i: (i, 0))
    return pl.pallas_call(
        _node_step_body,
        grid=(N // BLK_N,),
        in_specs=[row, row, row,
                  _full((D, D)), _full((D, D)), _full((1, D)),
                  _full((D, D)), _full((1, D)),
                  _full((D, D)), _full((1, D)),
                  _full((1, D)), _full((1, D))],
        out_specs=row,
        out_shape=jax.ShapeDtypeStruct((N, D), _F32),
    )(node_lat, a0, a1, wna, wnb, b1.reshape(1, D), w2, b2.reshape(1, D),
      w3, b3.reshape(1, D), lng.reshape(1, D), lnb.reshape(1, D))


def _dec_body(x, w1, b1, w2, b2, w3, b3, out):
    h = jnp.maximum(_dot(x[...], w1[...]) + b1[...], 0.0)
    h = jnp.maximum(_dot(h, w2[...]) + b2[...], 0.0)
    out[...] = _dot(h, w3[...]) + b3[...]


def _decoder(node_lat, mlp):
    (w1, b1), (w2, b2), (w3, b3) = mlp
    dout = w3.shape[1]
    w3p = jnp.zeros((D, D), _F32).at[:, :dout].set(w3)
    b3p = jnp.zeros((1, D), _F32).at[0, :dout].set(b3)
    row = pl.BlockSpec((BLK_N, D), lambda i: (i, 0))
    full = pl.pallas_call(
        _dec_body,
        grid=(N // BLK_N,),
        in_specs=[row,
                  _full((D, D)), _full((1, D)),
                  _full((D, D)), _full((1, D)),
                  _full((D, D)), _full((1, D))],
        out_specs=row,
        out_shape=jax.ShapeDtypeStruct((N, D), _F32),
    )(node_lat, w1, b1.reshape(1, D), w2, b2.reshape(1, D), w3p, b3p)
    return full[:, :dout]


def _sc_gather_body(ps_hbm, pr_hbm, snd_hbm, rcv_hbm, gs_hbm, gr_hbm,
                    idx_s, idx_r, rows_s, rows_r, sem_s, sem_r):
    cid = lax.axis_index("c")
    sid = lax.axis_index("s")
    base = cid * (EP // NC) + sid * PER_TILE

    def body(j, carry):
        off = base + j * GCH
        pltpu.sync_copy(snd_hbm.at[pl.ds(off, GCH)], idx_s)
        pltpu.sync_copy(rcv_hbm.at[pl.ds(off, GCH)], idx_r)
        a = pltpu.async_copy(ps_hbm.at[idx_s], rows_s, sem_s)
        b = pltpu.async_copy(pr_hbm.at[idx_r], rows_r, sem_r)
        a.wait()
        b.wait()
        pltpu.sync_copy(rows_s, gs_hbm.at[pl.ds(off, GCH)])
        pltpu.sync_copy(rows_r, gr_hbm.at[pl.ds(off, GCH)])
        return carry

    lax.fori_loop(0, GITERS, body, 0)


def _sc_scatter_body(ne_hbm, rcv_hbm, zeros_hbm, out_hbm, idx_v, rows_v,
                     agg_sh, sem):
    cid = lax.axis_index("c")
    sid = lax.axis_index("s")
    rpt = NPAD // NS
    pltpu.sync_copy(zeros_hbm.at[pl.ds(sid * rpt, rpt)],
                    agg_sh.at[pl.ds(sid * rpt, rpt)])
    plsc.subcore_barrier()
    base = cid * (EP // NC) + sid * PER_TILE

    def body(j, carry):
        off = base + j * GCH
        pltpu.sync_copy(rcv_hbm.at[pl.ds(off, GCH)], idx_v)
        pltpu.sync_copy(ne_hbm.at[pl.ds(off, GCH)], rows_v)
        pltpu.sync_copy(rows_v, agg_sh.at[idx_v], add=True)
        return carry

    lax.fori_loop(0, GITERS, body, 0)
    plsc.subcore_barrier()
    pltpu.sync_copy(agg_sh.at[pl.ds(sid * rpt, rpt)],
                    out_hbm.at[pl.ds(cid * NPAD + sid * rpt, rpt)])


@functools.cache
def _sc_calls():
    mesh = plsc.VectorSubcoreMesh(core_axis_name="c", subcore_axis_name="s")
    gather = pl.kernel(
        _sc_gather_body,
        out_type=(jax.ShapeDtypeStruct((EP, D), _F32),
                  jax.ShapeDtypeStruct((EP, D), _F32)),
        mesh=mesh,
        scratch_types=[
            pltpu.VMEM((GCH,), jnp.int32),
            pltpu.VMEM((GCH,), jnp.int32),
            pltpu.VMEM((GCH, D), _F32),
            pltpu.VMEM((GCH, D), _F32),
            pltpu.SemaphoreType.DMA,
            pltpu.SemaphoreType.DMA,
        ],
    )
    scatter = pl.kernel(
        _sc_scatter_body,
        out_type=jax.ShapeDtypeStruct((2 * NPAD, D), _F32),
        mesh=mesh,
        scratch_types=[
            pltpu.VMEM((GCH,), jnp.int32),
            pltpu.VMEM((GCH, D), _F32),
            pltpu.VMEM_SHARED((NPAD, D), _F32),
            pltpu.SemaphoreType.DMA,
        ],
    )
    return gather, scatter


def _gather_on_sc(ps, pr, snd, rcv):
    return _sc_calls()[0](ps, pr, snd, rcv)


def _scatter_on_sc(ne, rcv, zeros):
    return _sc_calls()[1](ne, rcv, zeros)


def kernel(node_features, edge_features, senders, receivers, params):
    p = params

    pad = EP - E
    senders_p = jnp.concatenate([senders, jnp.zeros((pad,), jnp.int32)])
    recv_gather_p = jnp.concatenate([receivers, jnp.zeros((pad,), jnp.int32)])
    recv_scatter_p = jnp.concatenate(
        [receivers, jnp.full((pad,), N, jnp.int32)])
    ef_p = jnp.concatenate(
        [edge_features, jnp.zeros((pad, edge_features.shape[1]), _F32)])
    zeros_acc = jnp.zeros((NPAD, D), _F32)

    node_lat = _mlp3_ln(node_features, p['enc_node'], *p['enc_node_ln'],
                        blk=BLK_N)
    edge_lat = _mlp3_ln(ef_p, p['enc_edge'], *p['enc_edge_ln'], blk=BLK_E)

    for sp in p['steps']:
        (w1, b1), (w2, b2), (w3, b3) = sp['edge_mlp']
        ws, wr, we = w1[:D], w1[D:2 * D], w1[2 * D:]
        ps, pr = _proj2(node_lat, ws, wr)
        gs, gr = _gather_on_sc(ps, pr, senders_p, recv_gather_p)
        new_e, edge_lat = _edge_step(gs, gr, edge_lat, we, b1, w2, b2, w3, b3,
                                     *sp['edge_ln'])
        aggs = _scatter_on_sc(new_e, recv_scatter_p, zeros_acc)
        (wn1, bn1), (wn2, bn2), (wn3, bn3) = sp['node_mlp']
        node_lat = _node_step(node_lat, aggs[:N], aggs[NPAD:NPAD + N],
                              wn1[:D], wn1[D:], bn1, wn2, bn2, wn3, bn3,
                              *sp['node_ln'])

    return _decoder(node_lat, p['dec'])


# speedup vs baseline: 2.0908x; 1.0294x over previous
import functools

import jax
import jax.numpy as jnp
from jax import lax
from jax.experimental import pallas as pl
from jax.experimental.pallas import tpu as pltpu
from jax.experimental.pallas import tpu_sc as plsc

N = 10000
E = 320000
D = 128
EP = 327680
NPAD = 10240
NC = 2
NS = 16
PER_TILE = EP // (NC * NS)
GCH = 128
GITERS = PER_TILE // GCH

BLK_E = 2048
BLK_N = 2000

_F32 = jnp.float32


def _dot(a, b):
    return jnp.dot(a, b, preferred_element_type=_F32)


def _ln(x, g, b):
    m = jnp.mean(x, axis=-1, keepdims=True)
    xc = x - m
    v = jnp.mean(xc * xc, axis=-1, keepdims=True)
    return xc * lax.rsqrt(v + 1e-5) * g + b


def _full(shape):
    nd = len(shape)
    return pl.BlockSpec(shape, lambda i: (0,) * nd)


def _mlp3_ln_body(x, w1, b1, w2, b2, w3, b3, g, b, out):
    h = jnp.maximum(_dot(x[...], w1[...]) + b1[...], 0.0)
    h = jnp.maximum(_dot(h, w2[...]) + b2[...], 0.0)
    h = _dot(h, w3[...]) + b3[...]
    out[...] = _ln(h, g[...], b[...])


def _mlp3_ln(x, mlp, lng, lnb, blk):
    (w1, b1), (w2, b2), (w3, b3) = mlp
    n, din = x.shape
    return pl.pallas_call(
        _mlp3_ln_body,
        grid=(n // blk,),
        in_specs=[
            pl.BlockSpec((blk, din), lambda i: (i, 0)),
            _full((din, D)), _full((1, D)),
            _full((D, D)), _full((1, D)),
            _full((D, D)), _full((1, D)),
            _full((1, D)), _full((1, D)),
        ],
        out_specs=pl.BlockSpec((blk, D), lambda i: (i, 0)),
        out_shape=jax.ShapeDtypeStruct((n, D), _F32),
    )(x, w1, b1.reshape(1, D), w2, b2.reshape(1, D), w3, b3.reshape(1, D),
      lng.reshape(1, D), lnb.reshape(1, D))


def _proj2_body(nl, ws, wr, ps_out, pr_out):
    x = nl[...]
    ps_out[...] = _dot(x, ws[...])
    pr_out[...] = _dot(x, wr[...])


def _proj2(node_lat, ws, wr):
    return pl.pallas_call(
        _proj2_body,
        grid=(N // BLK_N,),
        in_specs=[
            pl.BlockSpec((BLK_N, D), lambda i: (i, 0)),
            _full((D, D)), _full((D, D)),
        ],
        out_specs=[pl.BlockSpec((BLK_N, D), lambda i: (i, 0))] * 2,
        out_shape=[jax.ShapeDtypeStruct((N, D), _F32)] * 2,
    )(node_lat, ws, wr)


def _edge_step_body(gs, gr, el, w1e, b1, w2, b2, w3, b3, g, b, ne_out, el_out):
    x = gs[...] + gr[...] + _dot(el[...], w1e[...]) + b1[...]
    x = jnp.maximum(x, 0.0)
    x = jnp.maximum(_dot(x, w2[...]) + b2[...], 0.0)
    x = _dot(x, w3[...]) + b3[...]
    ne = _ln(x, g[...], b[...])
    ne_out[...] = ne
    el_out[...] = el[...] + ne


def _edge_step(gs, gr, el, w1e, b1, w2, b2, w3, b3, lng, lnb):
    row = pl.BlockSpec((BLK_E, D), lambda i: (i, 0))
    return pl.pallas_call(
        _edge_step_body,
        grid=(EP // BLK_E,),
        in_specs=[row, row, row,
                  _full((D, D)), _full((1, D)),
                  _full((D, D)), _full((1, D)),
                  _full((D, D)), _full((1, D)),
                  _full((1, D)), _full((1, D))],
        out_specs=[row, row],
        out_shape=[jax.ShapeDtypeStruct((EP, D), _F32)] * 2,
    )(gs, gr, el, w1e, b1.reshape(1, D), w2, b2.reshape(1, D),
      w3, b3.reshape(1, D), lng.reshape(1, D), lnb.reshape(1, D))


def _node_step_body(nl, a0, a1, wna, wnb, b1, w2, b2, w3, b3, g, b, out):
    nlv = nl[...]
    agg = a0[...] + a1[...]
    x = _dot(nlv, wna[...]) + _dot(agg, wnb[...]) + b1[...]
    x = jnp.maximum(x, 0.0)
    x = jnp.maximum(_dot(x, w2[...]) + b2[...], 0.0)
    x = _dot(x, w3[...]) + b3[...]
    out[...] = nlv + _ln(x, g[...], b[...])


def _node_step(node_lat, a0, a1, wna, wnb, b1, w2, b2, w3, b3, lng, lnb):
    row = pl.BlockSpec((BLK_N, D), lambda i: (i, 0))
    return pl.pallas_call(
        _node_step_body,
        grid=(N // BLK_N,),
        in_specs=[row, row, row,
                  _full((D, D)), _full((D, D)), _full((1, D)),
                  _full((D, D)), _full((1, D)),
                  _full((D, D)), _full((1, D)),
                  _full((1, D)), _full((1, D))],
        out_specs=row,
        out_shape=jax.ShapeDtypeStruct((N, D), _F32),
    )(node_lat, a0, a1, wna, wnb, b1.reshape(1, D), w2, b2.reshape(1, D),
      w3, b3.reshape(1, D), lng.reshape(1, D), lnb.reshape(1, D))


def _dec_body(x, w1, b1, w2, b2, w3, b3, out):
    h = jnp.maximum(_dot(x[...], w1[...]) + b1[...], 0.0)
    h = jnp.maximum(_dot(h, w2[...]) + b2[...], 0.0)
    out[...] = _dot(h, w3[...]) + b3[...]


def _decoder(node_lat, mlp):
    (w1, b1), (w2, b2), (w3, b3) = mlp
    dout = w3.shape[1]
    w3p = jnp.zeros((D, D), _F32).at[:, :dout].set(w3)
    b3p = jnp.zeros((1, D), _F32).at[0, :dout].set(b3)
    row = pl.BlockSpec((BLK_N, D), lambda i: (i, 0))
    full = pl.pallas_call(
        _dec_body,
        grid=(N // BLK_N,),
        in_specs=[row,
                  _full((D, D)), _full((1, D)),
                  _full((D, D)), _full((1, D)),
                  _full((D, D)), _full((1, D))],
        out_specs=row,
        out_shape=jax.ShapeDtypeStruct((N, D), _F32),
    )(node_lat, w1, b1.reshape(1, D), w2, b2.reshape(1, D), w3p, b3p)
    return full[:, :dout]


def _sc_gather_body(ps_hbm, pr_hbm, snd_hbm, rcv_hbm, gs_hbm, gr_hbm,
                    idx_s, idx_r, rows_s, rows_r,
                    sem_i0, sem_i1, sem_g, sem_w0, sem_w1):
    cid = lax.axis_index("c")
    sid = lax.axis_index("s")
    t0 = cid * ((EP // NC) // GCH) + sid * (PER_TILE // GCH)
    base = t0 * GCH
    sem_i = (sem_i0, sem_i1)
    sem_w = (sem_w0, sem_w1)

    pltpu.async_copy(snd_hbm.at[t0], idx_s.at[0], sem_i0)
    pltpu.async_copy(rcv_hbm.at[t0], idx_r.at[0], sem_i0)

    def outer(g, carry):
        for b in range(2):
            nb = 1 - b
            c = 2 * g + b
            off = base + c * GCH
            pltpu.make_async_copy(snd_hbm.at[t0 + c], idx_s.at[b],
                                  sem_i[b]).wait()
            pltpu.make_async_copy(rcv_hbm.at[t0 + c], idx_r.at[b],
                                  sem_i[b]).wait()
            @pl.when(c >= 2)
            def _():
                poff = off - 2 * GCH
                pltpu.make_async_copy(rows_s.at[b],
                                      gs_hbm.at[pl.ds(poff, GCH)],
                                      sem_w[b]).wait()
                pltpu.make_async_copy(rows_r.at[b],
                                      gr_hbm.at[pl.ds(poff, GCH)],
                                      sem_w[b]).wait()
            a = pltpu.async_copy(ps_hbm.at[idx_s.at[b]], rows_s.at[b], sem_g)
            d = pltpu.async_copy(pr_hbm.at[idx_r.at[b]], rows_r.at[b], sem_g)
            @pl.when(c + 1 < GITERS)
            def _():
                pltpu.async_copy(snd_hbm.at[t0 + c + 1], idx_s.at[nb],
                                 sem_i[nb])
                pltpu.async_copy(rcv_hbm.at[t0 + c + 1], idx_r.at[nb],
                                 sem_i[nb])
            a.wait()
            d.wait()
            pltpu.async_copy(rows_s.at[b], gs_hbm.at[pl.ds(off, GCH)],
                             sem_w[b])
            pltpu.async_copy(rows_r.at[b], gr_hbm.at[pl.ds(off, GCH)],
                             sem_w[b])
        return carry

    lax.fori_loop(0, GITERS // 2, outer, 0)
    for b in range(2):
        off = base + (GITERS - 2 + b) * GCH
        pltpu.make_async_copy(rows_s.at[b], gs_hbm.at[pl.ds(off, GCH)],
                              sem_w[b]).wait()
        pltpu.make_async_copy(rows_r.at[b], gr_hbm.at[pl.ds(off, GCH)],
                              sem_w[b]).wait()


def _sc_scatter_body(ne_hbm, rcv_hbm, zeros_hbm, out_hbm, idx_v, rows_v,
                     agg_sh, sem_l0, sem_l1, sem_s0, sem_s1):
    cid = lax.axis_index("c")
    sid = lax.axis_index("s")
    rpt = NPAD // NS
    pltpu.sync_copy(zeros_hbm.at[pl.ds(sid * rpt, rpt)],
                    agg_sh.at[pl.ds(sid * rpt, rpt)])
    plsc.subcore_barrier()
    t0 = cid * ((EP // NC) // GCH) + sid * (PER_TILE // GCH)
    base = t0 * GCH
    sem_l = (sem_l0, sem_l1)
    sem_s = (sem_s0, sem_s1)

    pltpu.async_copy(rcv_hbm.at[t0], idx_v.at[0], sem_l0)
    pltpu.async_copy(ne_hbm.at[pl.ds(base, GCH)], rows_v.at[0], sem_l0)

    def outer(g, carry):
        for b in range(2):
            nb = 1 - b
            c = 2 * g + b
            off = base + c * GCH
            pltpu.make_async_copy(rcv_hbm.at[t0 + c], idx_v.at[b],
                                  sem_l[b]).wait()
            pltpu.make_async_copy(ne_hbm.at[pl.ds(off, GCH)], rows_v.at[b],
                                  sem_l[b]).wait()
            pltpu.async_copy(rows_v.at[b], agg_sh.at[idx_v.at[b]], sem_s[b],
                             add=True)
            @pl.when(c >= 1)
            def _():
                pltpu.make_async_copy(rows_v.at[nb],
                                      agg_sh.at[idx_v.at[nb]],
                                      sem_s[nb]).wait()
            @pl.when(c + 1 < GITERS)
            def _():
                noff = off + GCH
                pltpu.async_copy(rcv_hbm.at[t0 + c + 1], idx_v.at[nb],
                                 sem_l[nb])
                pltpu.async_copy(ne_hbm.at[pl.ds(noff, GCH)], rows_v.at[nb],
                                 sem_l[nb])
        return carry

    lax.fori_loop(0, GITERS // 2, outer, 0)
    pltpu.make_async_copy(rows_v.at[1], agg_sh.at[idx_v.at[1]],
                          sem_s[1]).wait()
    plsc.subcore_barrier()
    pltpu.sync_copy(agg_sh.at[pl.ds(sid * rpt, rpt)],
                    out_hbm.at[pl.ds(cid * NPAD + sid * rpt, rpt)])


@functools.cache
def _sc_calls():
    mesh = plsc.VectorSubcoreMesh(core_axis_name="c", subcore_axis_name="s")
    gather = pl.kernel(
        _sc_gather_body,
        out_type=(jax.ShapeDtypeStruct((EP, D), _F32),
                  jax.ShapeDtypeStruct((EP, D), _F32)),
        mesh=mesh,
        scratch_types=[
            pltpu.VMEM((2, GCH), jnp.int32),
            pltpu.VMEM((2, GCH), jnp.int32),
            pltpu.VMEM((2, GCH, D), _F32),
            pltpu.VMEM((2, GCH, D), _F32),
            pltpu.SemaphoreType.DMA,
            pltpu.SemaphoreType.DMA,
            pltpu.SemaphoreType.DMA,
            pltpu.SemaphoreType.DMA,
            pltpu.SemaphoreType.DMA,
        ],
    )
    scatter = pl.kernel(
        _sc_scatter_body,
        out_type=jax.ShapeDtypeStruct((2 * NPAD, D), _F32),
        mesh=mesh,
        scratch_types=[
            pltpu.VMEM((2, GCH), jnp.int32),
            pltpu.VMEM((2, GCH, D), _F32),
            pltpu.VMEM_SHARED((NPAD, D), _F32),
            pltpu.SemaphoreType.DMA,
            pltpu.SemaphoreType.DMA,
            pltpu.SemaphoreType.DMA,
            pltpu.SemaphoreType.DMA,
        ],
    )
    return gather, scatter


def _gather_on_sc(ps, pr, snd, rcv):
    return _sc_calls()[0](ps, pr, snd, rcv)


def _scatter_on_sc(ne, rcv, zeros):
    return _sc_calls()[1](ne, rcv, zeros)


def kernel(node_features, edge_features, senders, receivers, params):
    p = params

    pad = EP - E
    senders_p = jnp.concatenate(
        [senders, jnp.zeros((pad,), jnp.int32)]).reshape(EP // GCH, GCH)
    recv_gather_p = jnp.concatenate(
        [receivers, jnp.zeros((pad,), jnp.int32)]).reshape(EP // GCH, GCH)
    recv_scatter_p = jnp.concatenate(
        [receivers, jnp.full((pad,), N, jnp.int32)]).reshape(EP // GCH, GCH)
    ef_p = jnp.concatenate(
        [edge_features, jnp.zeros((pad, edge_features.shape[1]), _F32)])
    zeros_acc = jnp.zeros((NPAD, D), _F32)

    node_lat = _mlp3_ln(node_features, p['enc_node'], *p['enc_node_ln'],
                        blk=BLK_N)
    edge_lat = _mlp3_ln(ef_p, p['enc_edge'], *p['enc_edge_ln'], blk=BLK_E)

    for sp in p['steps']:
        (w1, b1), (w2, b2), (w3, b3) = sp['edge_mlp']
        ws, wr, we = w1[:D], w1[D:2 * D], w1[2 * D:]
        ps, pr = _proj2(node_lat, ws, wr)
        gs, gr = _gather_on_sc(ps, pr, senders_p, recv_gather_p)
        new_e, edge_lat = _edge_step(gs, gr, edge_lat, we, b1, w2, b2, w3, b3,
                                     *sp['edge_ln'])
        aggs = _scatter_on_sc(new_e, recv_scatter_p, zeros_acc)
        (wn1, bn1), (wn2, bn2), (wn3, bn3) = sp['node_mlp']
        node_lat = _node_step(node_lat, aggs[:N], aggs[NPAD:NPAD + N],
                              wn1[:D], wn1[D:], bn1, wn2, bn2, wn3, bn3,
                              *sp['node_ln'])

    return _decoder(node_lat, p['dec'])


# speedup vs baseline: 4.2224x; 2.0195x over previous
import functools

import jax
import jax.numpy as jnp
from jax import lax
from jax.experimental import pallas as pl
from jax.experimental.pallas import tpu as pltpu
from jax.experimental.pallas import tpu_sc as plsc

N = 10000
E = 320000
D = 128
EP = 327680
NPAD = 10240
NC = 2
NS = 16
PER_TILE = EP // (NC * NS)
GCH = 128
GITERS = PER_TILE // GCH

BLK_E = 2048
BLK_N = 2000

_F32 = jnp.float32


def _dot(a, b):
    return jnp.dot(a, b, preferred_element_type=_F32)


def _ln(x, g, b):
    m = jnp.mean(x, axis=-1, keepdims=True)
    xc = x - m
    v = jnp.mean(xc * xc, axis=-1, keepdims=True)
    return xc * lax.rsqrt(v + 1e-5) * g + b


def _full(shape):
    nd = len(shape)
    return pl.BlockSpec(shape, lambda i: (0,) * nd)


def _mlp3_ln_body(x, w1, b1, w2, b2, w3, b3, g, b, out):
    h = jnp.maximum(_dot(x[...], w1[...]) + b1[...], 0.0)
    h = jnp.maximum(_dot(h, w2[...]) + b2[...], 0.0)
    h = _dot(h, w3[...]) + b3[...]
    out[...] = _ln(h, g[...], b[...])


def _mlp3_ln(x, mlp, lng, lnb, blk):
    (w1, b1), (w2, b2), (w3, b3) = mlp
    n, din = x.shape
    return pl.pallas_call(
        _mlp3_ln_body,
        grid=(n // blk,),
        in_specs=[
            pl.BlockSpec((blk, din), lambda i: (i, 0)),
            _full((din, D)), _full((1, D)),
            _full((D, D)), _full((1, D)),
            _full((D, D)), _full((1, D)),
            _full((1, D)), _full((1, D)),
        ],
        out_specs=pl.BlockSpec((blk, D), lambda i: (i, 0)),
        out_shape=jax.ShapeDtypeStruct((n, D), _F32),
    )(x, w1, b1.reshape(1, D), w2, b2.reshape(1, D), w3, b3.reshape(1, D),
      lng.reshape(1, D), lnb.reshape(1, D))


def _proj2_body(nl, ws, wr, ps_out, pr_out):
    x = nl[...]
    ps_out[...] = _dot(x, ws[...])
    pr_out[...] = _dot(x, wr[...])


def _proj2(node_lat, ws, wr):
    return pl.pallas_call(
        _proj2_body,
        grid=(N // BLK_N,),
        in_specs=[
            pl.BlockSpec((BLK_N, D), lambda i: (i, 0)),
            _full((D, D)), _full((D, D)),
        ],
        out_specs=[pl.BlockSpec((BLK_N, D), lambda i: (i, 0))] * 2,
        out_shape=[jax.ShapeDtypeStruct((N, D), _F32)] * 2,
    )(node_lat, ws, wr)


def _edge_step_body(gs, gr, el, w1e, b1, w2, b2, w3, b3, g, b, ne_out, el_out):
    x = gs[...] + gr[...] + _dot(el[...], w1e[...]) + b1[...]
    x = jnp.maximum(x, 0.0)
    x = jnp.maximum(_dot(x, w2[...]) + b2[...], 0.0)
    x = _dot(x, w3[...]) + b3[...]
    ne = _ln(x, g[...], b[...])
    ne_out[...] = ne
    el_out[...] = el[...] + ne


def _edge_step(gs, gr, el, w1e, b1, w2, b2, w3, b3, lng, lnb):
    row = pl.BlockSpec((BLK_E, D), lambda i: (i, 0))
    return pl.pallas_call(
        _edge_step_body,
        grid=(EP // BLK_E,),
        in_specs=[row, row, row,
                  _full((D, D)), _full((1, D)),
                  _full((D, D)), _full((1, D)),
                  _full((D, D)), _full((1, D)),
                  _full((1, D)), _full((1, D))],
        out_specs=[row, row],
        out_shape=[jax.ShapeDtypeStruct((EP, D), _F32)] * 2,
    )(gs, gr, el, w1e, b1.reshape(1, D), w2, b2.reshape(1, D),
      w3, b3.reshape(1, D), lng.reshape(1, D), lnb.reshape(1, D))


def _node_step_body(nl, a0, a1, wna, wnb, b1, w2, b2, w3, b3, g, b, out):
    nlv = nl[...]
    agg = a0[...] + a1[...]
    x = _dot(nlv, wna[...]) + _dot(agg, wnb[...]) + b1[...]
    x = jnp.maximum(x, 0.0)
    x = jnp.maximum(_dot(x, w2[...]) + b2[...], 0.0)
    x = _dot(x, w3[...]) + b3[...]
    out[...] = nlv + _ln(x, g[...], b[...])


def _node_step(node_lat, a0, a1, wna, wnb, b1, w2, b2, w3, b3, lng, lnb):
    row = pl.BlockSpec((BLK_N, D), lambda i: (i, 0))
    return pl.pallas_call(
        _node_step_body,
        grid=(N // BLK_N,),
        in_specs=[row, row, row,
                  _full((D, D)), _full((D, D)), _full((1, D)),
                  _full((D, D)), _full((1, D)),
                  _full((D, D)), _full((1, D)),
                  _full((1, D)), _full((1, D))],
        out_specs=row,
        out_shape=jax.ShapeDtypeStruct((N, D), _F32),
    )(node_lat, a0, a1, wna, wnb, b1.reshape(1, D), w2, b2.reshape(1, D),
      w3, b3.reshape(1, D), lng.reshape(1, D), lnb.reshape(1, D))


def _dec_body(x, w1, b1, w2, b2, w3, b3, out):
    h = jnp.maximum(_dot(x[...], w1[...]) + b1[...], 0.0)
    h = jnp.maximum(_dot(h, w2[...]) + b2[...], 0.0)
    out[...] = _dot(h, w3[...]) + b3[...]


def _decoder(node_lat, mlp):
    (w1, b1), (w2, b2), (w3, b3) = mlp
    dout = w3.shape[1]
    w3p = jnp.zeros((D, D), _F32).at[:, :dout].set(w3)
    b3p = jnp.zeros((1, D), _F32).at[0, :dout].set(b3)
    row = pl.BlockSpec((BLK_N, D), lambda i: (i, 0))
    full = pl.pallas_call(
        _dec_body,
        grid=(N // BLK_N,),
        in_specs=[row,
                  _full((D, D)), _full((1, D)),
                  _full((D, D)), _full((1, D)),
                  _full((D, D)), _full((1, D))],
        out_specs=row,
        out_shape=jax.ShapeDtypeStruct((N, D), _F32),
    )(node_lat, w1, b1.reshape(1, D), w2, b2.reshape(1, D), w3p, b3p)
    return full[:, :dout]


TCHUNKS = (EP // GCH) // NS


def _sc_gather_body(ps_hbm, pr_hbm, snd_hbm, rcv_hbm, gs_hbm, gr_hbm,
                    tab_sh, idx, rows,
                    sem_i0, sem_i1, sem_g, sem_w0, sem_w1):
    cid = lax.axis_index("c")
    sid = lax.axis_index("s")

    def stage(tab_hbm):
        @pl.when(sid < NS - 1)
        def _():
            pltpu.sync_copy(tab_hbm.at[pl.ds(sid * 640, 640)],
                            tab_sh.at[pl.ds(sid * 640, 640)])

        @pl.when(sid == NS - 1)
        def _():
            pltpu.sync_copy(tab_hbm.at[pl.ds(9600, 400)],
                            tab_sh.at[pl.ds(9600, 400)])

    @pl.when(cid == 0)
    def _():
        stage(ps_hbm)

    @pl.when(cid == 1)
    def _():
        stage(pr_hbm)

    plsc.subcore_barrier()

    t0 = sid * TCHUNKS
    base = t0 * GCH
    sem_i = (sem_i0, sem_i1)
    sem_w = (sem_w0, sem_w1)

    def pipe(idx_hbm, out_hbm):
        pltpu.async_copy(idx_hbm.at[t0], idx.at[0], sem_i0)

        def outer(g, carry):
            for b in range(2):
                nb = 1 - b
                c = 2 * g + b
                off = base + c * GCH
                pltpu.make_async_copy(idx_hbm.at[t0 + c], idx.at[b],
                                      sem_i[b]).wait()
                @pl.when(c >= 2)
                def _():
                    poff = off - 2 * GCH
                    pltpu.make_async_copy(rows.at[b],
                                          out_hbm.at[pl.ds(poff, GCH)],
                                          sem_w[b]).wait()
                a = pltpu.async_copy(tab_sh.at[idx.at[b]], rows.at[b],
                                     sem_g)
                @pl.when(c + 1 < TCHUNKS)
                def _():
                    pltpu.async_copy(idx_hbm.at[t0 + c + 1], idx.at[nb],
                                     sem_i[nb])
                a.wait()
                pltpu.async_copy(rows.at[b], out_hbm.at[pl.ds(off, GCH)],
                                 sem_w[b])
            return carry

        lax.fori_loop(0, TCHUNKS // 2, outer, 0)
        for b in range(2):
            off = base + (TCHUNKS - 2 + b) * GCH
            pltpu.make_async_copy(rows.at[b], out_hbm.at[pl.ds(off, GCH)],
                                  sem_w[b]).wait()

    @pl.when(cid == 0)
    def _():
        pipe(snd_hbm, gs_hbm)

    @pl.when(cid == 1)
    def _():
        pipe(rcv_hbm, gr_hbm)


def _sc_scatter_body(ne_hbm, rcv_hbm, zeros_hbm, out_hbm, idx_v, rows_v,
                     agg_sh, sem_l0, sem_l1, sem_s0, sem_s1):
    cid = lax.axis_index("c")
    sid = lax.axis_index("s")
    rpt = NPAD // NS
    pltpu.sync_copy(zeros_hbm.at[pl.ds(sid * rpt, rpt)],
                    agg_sh.at[pl.ds(sid * rpt, rpt)])
    plsc.subcore_barrier()
    t0 = cid * ((EP // NC) // GCH) + sid * (PER_TILE // GCH)
    base = t0 * GCH
    sem_l = (sem_l0, sem_l1)
    sem_s = (sem_s0, sem_s1)

    pltpu.async_copy(rcv_hbm.at[t0], idx_v.at[0], sem_l0)
    pltpu.async_copy(ne_hbm.at[pl.ds(base, GCH)], rows_v.at[0], sem_l0)

    def outer(g, carry):
        for b in range(2):
            nb = 1 - b
            c = 2 * g + b
            off = base + c * GCH
            pltpu.make_async_copy(rcv_hbm.at[t0 + c], idx_v.at[b],
                                  sem_l[b]).wait()
            pltpu.make_async_copy(ne_hbm.at[pl.ds(off, GCH)], rows_v.at[b],
                                  sem_l[b]).wait()
            pltpu.async_copy(rows_v.at[b], agg_sh.at[idx_v.at[b]], sem_s[b],
                             add=True)
            @pl.when(c >= 1)
            def _():
                pltpu.make_async_copy(rows_v.at[nb],
                                      agg_sh.at[idx_v.at[nb]],
                                      sem_s[nb]).wait()
            @pl.when(c + 1 < GITERS)
            def _():
                noff = off + GCH
                pltpu.async_copy(rcv_hbm.at[t0 + c + 1], idx_v.at[nb],
                                 sem_l[nb])
                pltpu.async_copy(ne_hbm.at[pl.ds(noff, GCH)], rows_v.at[nb],
                                 sem_l[nb])
        return carry

    lax.fori_loop(0, GITERS // 2, outer, 0)
    pltpu.make_async_copy(rows_v.at[1], agg_sh.at[idx_v.at[1]],
                          sem_s[1]).wait()
    plsc.subcore_barrier()
    pltpu.sync_copy(agg_sh.at[pl.ds(sid * rpt, rpt)],
                    out_hbm.at[pl.ds(cid * NPAD + sid * rpt, rpt)])


@functools.cache
def _sc_calls():
    mesh = plsc.VectorSubcoreMesh(core_axis_name="c", subcore_axis_name="s")
    gather = pl.kernel(
        _sc_gather_body,
        out_type=(jax.ShapeDtypeStruct((EP, D), _F32),
                  jax.ShapeDtypeStruct((EP, D), _F32)),
        mesh=mesh,
        scratch_types=[
            pltpu.VMEM_SHARED((N, D), _F32),
            pltpu.VMEM((2, GCH), jnp.int32),
            pltpu.VMEM((2, GCH, D), _F32),
            pltpu.SemaphoreType.DMA,
            pltpu.SemaphoreType.DMA,
            pltpu.SemaphoreType.DMA,
            pltpu.SemaphoreType.DMA,
            pltpu.SemaphoreType.DMA,
        ],
    )
    scatter = pl.kernel(
        _sc_scatter_body,
        out_type=jax.ShapeDtypeStruct((2 * NPAD, D), _F32),
        mesh=mesh,
        scratch_types=[
            pltpu.VMEM((2, GCH), jnp.int32),
            pltpu.VMEM((2, GCH, D), _F32),
            pltpu.VMEM_SHARED((NPAD, D), _F32),
            pltpu.SemaphoreType.DMA,
            pltpu.SemaphoreType.DMA,
            pltpu.SemaphoreType.DMA,
            pltpu.SemaphoreType.DMA,
        ],
    )
    return gather, scatter


def _gather_on_sc(ps, pr, snd, rcv):
    return _sc_calls()[0](ps, pr, snd, rcv)


def _scatter_on_sc(ne, rcv, zeros):
    return _sc_calls()[1](ne, rcv, zeros)


def kernel(node_features, edge_features, senders, receivers, params):
    p = params

    pad = EP - E
    senders_p = jnp.concatenate(
        [senders, jnp.zeros((pad,), jnp.int32)]).reshape(EP // GCH, GCH)
    recv_gather_p = jnp.concatenate(
        [receivers, jnp.zeros((pad,), jnp.int32)]).reshape(EP // GCH, GCH)
    recv_scatter_p = jnp.concatenate(
        [receivers, jnp.full((pad,), N, jnp.int32)]).reshape(EP // GCH, GCH)
    ef_p = jnp.concatenate(
        [edge_features, jnp.zeros((pad, edge_features.shape[1]), _F32)])
    zeros_acc = jnp.zeros((NPAD, D), _F32)

    node_lat = _mlp3_ln(node_features, p['enc_node'], *p['enc_node_ln'],
                        blk=BLK_N)
    edge_lat = _mlp3_ln(ef_p, p['enc_edge'], *p['enc_edge_ln'], blk=BLK_E)

    for sp in p['steps']:
        (w1, b1), (w2, b2), (w3, b3) = sp['edge_mlp']
        ws, wr, we = w1[:D], w1[D:2 * D], w1[2 * D:]
        ps, pr = _proj2(node_lat, ws, wr)
        gs, gr = _gather_on_sc(ps, pr, senders_p, recv_gather_p)
        new_e, edge_lat = _edge_step(gs, gr, edge_lat, we, b1, w2, b2, w3, b3,
                                     *sp['edge_ln'])
        aggs = _scatter_on_sc(new_e, recv_scatter_p, zeros_acc)
        (wn1, bn1), (wn2, bn2), (wn3, bn3) = sp['node_mlp']
        node_lat = _node_step(node_lat, aggs[:N], aggs[NPAD:NPAD + N],
                              wn1[:D], wn1[D:], bn1, wn2, bn2, wn3, bn3,
                              *sp['node_ln'])

    return _decoder(node_lat, p['dec'])


# speedup vs baseline: 4.3640x; 1.0335x over previous
import functools

import jax
import jax.numpy as jnp
from jax import lax
from jax.experimental import pallas as pl
from jax.experimental.pallas import tpu as pltpu
from jax.experimental.pallas import tpu_sc as plsc

N = 10000
E = 320000
D = 128
EP = 327680
EPH = EP // 2
NPAD = 10240
NC = 2
NS = 16
GCH = 128

BLK_E = 2048
BLK_N = 2000

_F32 = jnp.float32


def _dot(a, b):
    return jnp.dot(a, b, preferred_element_type=_F32)


def _ln(x, g, b):
    m = jnp.mean(x, axis=-1, keepdims=True)
    xc = x - m
    v = jnp.mean(xc * xc, axis=-1, keepdims=True)
    return xc * lax.rsqrt(v + 1e-5) * g + b


def _full(shape):
    nd = len(shape)
    return pl.BlockSpec(shape, lambda i: (0,) * nd)


def _mlp3_ln_body(x, w1, b1, w2, b2, w3, b3, g, b, out):
    h = jnp.maximum(_dot(x[...], w1[...]) + b1[...], 0.0)
    h = jnp.maximum(_dot(h, w2[...]) + b2[...], 0.0)
    h = _dot(h, w3[...]) + b3[...]
    out[...] = _ln(h, g[...], b[...])


def _mlp3_ln(x, mlp, lng, lnb, blk):
    (w1, b1), (w2, b2), (w3, b3) = mlp
    n, din = x.shape
    return pl.pallas_call(
        _mlp3_ln_body,
        grid=(n // blk,),
        in_specs=[
            pl.BlockSpec((blk, din), lambda i: (i, 0)),
            _full((din, D)), _full((1, D)),
            _full((D, D)), _full((1, D)),
            _full((D, D)), _full((1, D)),
            _full((1, D)), _full((1, D)),
        ],
        out_specs=pl.BlockSpec((blk, D), lambda i: (i, 0)),
        out_shape=jax.ShapeDtypeStruct((n, D), _F32),
    )(x, w1, b1.reshape(1, D), w2, b2.reshape(1, D), w3, b3.reshape(1, D),
      lng.reshape(1, D), lnb.reshape(1, D))


def _proj2_body(nl, ws, wr, ps_out, pr_out):
    x = nl[...]
    ps_out[...] = _dot(x, ws[...])
    pr_out[...] = _dot(x, wr[...])


def _proj2(node_lat, ws, wr):
    return pl.pallas_call(
        _proj2_body,
        grid=(N // BLK_N,),
        in_specs=[
            pl.BlockSpec((BLK_N, D), lambda i: (i, 0)),
            _full((D, D)), _full((D, D)),
        ],
        out_specs=[pl.BlockSpec((BLK_N, D), lambda i: (i, 0))] * 2,
        out_shape=[jax.ShapeDtypeStruct((N, D), _F32)] * 2,
    )(node_lat, ws, wr)


def _edge_step_body(gs, gr, el, w1e, b1, w2, b2, w3, b3, g, b, ne_out, el_out):
    x = gs[...] + gr[...] + _dot(el[...], w1e[...]) + b1[...]
    x = jnp.maximum(x, 0.0)
    x = jnp.maximum(_dot(x, w2[...]) + b2[...], 0.0)
    x = _dot(x, w3[...]) + b3[...]
    ne = _ln(x, g[...], b[...])
    ne_out[...] = ne
    el_out[...] = el[...] + ne


def _edge_step(gs, gr, el, w1e, b1, w2, b2, w3, b3, lng, lnb):
    n = gs.shape[0]
    row = pl.BlockSpec((BLK_E, D), lambda i: (i, 0))
    return pl.pallas_call(
        _edge_step_body,
        grid=(n // BLK_E,),
        in_specs=[row, row, row,
                  _full((D, D)), _full((1, D)),
                  _full((D, D)), _full((1, D)),
                  _full((D, D)), _full((1, D)),
                  _full((1, D)), _full((1, D))],
        out_specs=[row, row],
        out_shape=[jax.ShapeDtypeStruct((n, D), _F32)] * 2,
    )(gs, gr, el, w1e, b1.reshape(1, D), w2, b2.reshape(1, D),
      w3, b3.reshape(1, D), lng.reshape(1, D), lnb.reshape(1, D))


def _node_step_body(nl, a0, a1, a2, a3, wna, wnb, b1, w2, b2, w3, b3, g, b,
                    out):
    nlv = nl[...]
    agg = (a0[...] + a1[...]) + (a2[...] + a3[...])
    x = _dot(nlv, wna[...]) + _dot(agg, wnb[...]) + b1[...]
    x = jnp.maximum(x, 0.0)
    x = jnp.maximum(_dot(x, w2[...]) + b2[...], 0.0)
    x = _dot(x, w3[...]) + b3[...]
    out[...] = nlv + _ln(x, g[...], b[...])


def _node_step(node_lat, a0, a1, a2, a3, wna, wnb, b1, w2, b2, w3, b3,
               lng, lnb):
    row = pl.BlockSpec((BLK_N, D), lambda i: (i, 0))
    return pl.pallas_call(
        _node_step_body,
        grid=(N // BLK_N,),
        in_specs=[row, row, row, row, row,
                  _full((D, D)), _full((D, D)), _full((1, D)),
                  _full((D, D)), _full((1, D)),
                  _full((D, D)), _full((1, D)),
                  _full((1, D)), _full((1, D))],
        out_specs=row,
        out_shape=jax.ShapeDtypeStruct((N, D), _F32),
    )(node_lat, a0, a1, a2, a3, wna, wnb, b1.reshape(1, D),
      w2, b2.reshape(1, D), w3, b3.reshape(1, D), lng.reshape(1, D),
      lnb.reshape(1, D))


def _dec_body(x, w1, b1, w2, b2, w3, b3, out):
    h = jnp.maximum(_dot(x[...], w1[...]) + b1[...], 0.0)
    h = jnp.maximum(_dot(h, w2[...]) + b2[...], 0.0)
    out[...] = _dot(h, w3[...]) + b3[...]


def _decoder(node_lat, mlp):
    (w1, b1), (w2, b2), (w3, b3) = mlp
    dout = w3.shape[1]
    w3p = jnp.zeros((D, D), _F32).at[:, :dout].set(w3)
    b3p = jnp.zeros((1, D), _F32).at[0, :dout].set(b3)
    row = pl.BlockSpec((BLK_N, D), lambda i: (i, 0))
    full = pl.pallas_call(
        _dec_body,
        grid=(N // BLK_N,),
        in_specs=[row,
                  _full((D, D)), _full((1, D)),
                  _full((D, D)), _full((1, D)),
                  _full((D, D)), _full((1, D))],
        out_specs=row,
        out_shape=jax.ShapeDtypeStruct((N, D), _F32),
    )(node_lat, w1, b1.reshape(1, D), w2, b2.reshape(1, D), w3p, b3p)
    return full[:, :dout]


TCHUNKS = (EPH // GCH) // NS
SITERS = (EPH // (NC * NS)) // GCH


def _sc_gather_body(ps_hbm, pr_hbm, snd_hbm, rcv_hbm, gs_hbm, gr_hbm,
                    tab_sh, idx, rows,
                    sem_i0, sem_i1, sem_g, sem_w0, sem_w1):
    cid = lax.axis_index("c")
    sid = lax.axis_index("s")

    def stage(tab_hbm):
        @pl.when(sid < NS - 1)
        def _():
            pltpu.sync_copy(tab_hbm.at[pl.ds(sid * 640, 640)],
                            tab_sh.at[pl.ds(sid * 640, 640)])

        @pl.when(sid == NS - 1)
        def _():
            pltpu.sync_copy(tab_hbm.at[pl.ds(9600, 400)],
                            tab_sh.at[pl.ds(9600, 400)])

    @pl.when(cid == 0)
    def _():
        stage(ps_hbm)

    @pl.when(cid == 1)
    def _():
        stage(pr_hbm)

    plsc.subcore_barrier()

    t0 = sid * TCHUNKS
    base = t0 * GCH
    sem_i = (sem_i0, sem_i1)
    sem_w = (sem_w0, sem_w1)

    def pipe(idx_hbm, out_hbm):
        pltpu.async_copy(idx_hbm.at[t0], idx.at[0], sem_i0)

        def outer(g, carry):
            for b in range(2):
                nb = 1 - b
                c = 2 * g + b
                off = base + c * GCH
                pltpu.make_async_copy(idx_hbm.at[t0 + c], idx.at[b],
                                      sem_i[b]).wait()
                @pl.when(c >= 2)
                def _():
                    poff = off - 2 * GCH
                    pltpu.make_async_copy(rows.at[b],
                                          out_hbm.at[pl.ds(poff, GCH)],
                                          sem_w[b]).wait()
                a = pltpu.async_copy(tab_sh.at[idx.at[b]], rows.at[b],
                                     sem_g)
                @pl.when(c + 1 < TCHUNKS)
                def _():
                    pltpu.async_copy(idx_hbm.at[t0 + c + 1], idx.at[nb],
                                     sem_i[nb])
                a.wait()
                pltpu.async_copy(rows.at[b], out_hbm.at[pl.ds(off, GCH)],
                                 sem_w[b])
            return carry

        lax.fori_loop(0, TCHUNKS // 2, outer, 0)
        for b in range(2):
            off = base + (TCHUNKS - 2 + b) * GCH
            pltpu.make_async_copy(rows.at[b], out_hbm.at[pl.ds(off, GCH)],
                                  sem_w[b]).wait()

    @pl.when(cid == 0)
    def _():
        pipe(snd_hbm, gs_hbm)

    @pl.when(cid == 1)
    def _():
        pipe(rcv_hbm, gr_hbm)


def _sc_scatter_body(ne_hbm, rcv_hbm, zeros_hbm, out_hbm, idx_v, rows_v,
                     agg_sh, sem_l0, sem_l1, sem_s0, sem_s1):
    cid = lax.axis_index("c")
    sid = lax.axis_index("s")
    rpt = NPAD // NS
    pltpu.sync_copy(zeros_hbm.at[pl.ds(sid * rpt, rpt)],
                    agg_sh.at[pl.ds(sid * rpt, rpt)])
    plsc.subcore_barrier()
    t0 = cid * ((EPH // NC) // GCH) + sid * SITERS
    base = t0 * GCH
    sem_l = (sem_l0, sem_l1)
    sem_s = (sem_s0, sem_s1)

    pltpu.async_copy(rcv_hbm.at[t0], idx_v.at[0], sem_l0)
    pltpu.async_copy(ne_hbm.at[pl.ds(base, GCH)], rows_v.at[0], sem_l0)

    def outer(g, carry):
        for b in range(2):
            nb = 1 - b
            c = 2 * g + b
            off = base + c * GCH
            pltpu.make_async_copy(rcv_hbm.at[t0 + c], idx_v.at[b],
                                  sem_l[b]).wait()
            pltpu.make_async_copy(ne_hbm.at[pl.ds(off, GCH)], rows_v.at[b],
                                  sem_l[b]).wait()
            pltpu.async_copy(rows_v.at[b], agg_sh.at[idx_v.at[b]], sem_s[b],
                             add=True)
            @pl.when(c >= 1)
            def _():
                pltpu.make_async_copy(rows_v.at[nb],
                                      agg_sh.at[idx_v.at[nb]],
                                      sem_s[nb]).wait()
            @pl.when(c + 1 < SITERS)
            def _():
                noff = off + GCH
                pltpu.async_copy(rcv_hbm.at[t0 + c + 1], idx_v.at[nb],
                                 sem_l[nb])
                pltpu.async_copy(ne_hbm.at[pl.ds(noff, GCH)], rows_v.at[nb],
                                 sem_l[nb])
        return carry

    lax.fori_loop(0, SITERS // 2, outer, 0)
    pltpu.make_async_copy(rows_v.at[1], agg_sh.at[idx_v.at[1]],
                          sem_s[1]).wait()
    plsc.subcore_barrier()
    pltpu.sync_copy(agg_sh.at[pl.ds(sid * rpt, rpt)],
                    out_hbm.at[pl.ds(cid * NPAD + sid * rpt, rpt)])


@functools.cache
def _sc_calls():
    mesh = plsc.VectorSubcoreMesh(core_axis_name="c", subcore_axis_name="s")
    gather = pl.kernel(
        _sc_gather_body,
        out_type=(jax.ShapeDtypeStruct((EPH, D), _F32),
                  jax.ShapeDtypeStruct((EPH, D), _F32)),
        mesh=mesh,
        scratch_types=[
            pltpu.VMEM_SHARED((N, D), _F32),
            pltpu.VMEM((2, GCH), jnp.int32),
            pltpu.VMEM((2, GCH, D), _F32),
            pltpu.SemaphoreType.DMA,
            pltpu.SemaphoreType.DMA,
            pltpu.SemaphoreType.DMA,
            pltpu.SemaphoreType.DMA,
            pltpu.SemaphoreType.DMA,
        ],
    )
    scatter = pl.kernel(
        _sc_scatter_body,
        out_type=jax.ShapeDtypeStruct((2 * NPAD, D), _F32),
        mesh=mesh,
        scratch_types=[
            pltpu.VMEM((2, GCH), jnp.int32),
            pltpu.VMEM((2, GCH, D), _F32),
            pltpu.VMEM_SHARED((NPAD, D), _F32),
            pltpu.SemaphoreType.DMA,
            pltpu.SemaphoreType.DMA,
            pltpu.SemaphoreType.DMA,
            pltpu.SemaphoreType.DMA,
        ],
    )
    return gather, scatter


def _gather_on_sc(ps, pr, snd, rcv):
    return _sc_calls()[0](ps, pr, snd, rcv)


def _scatter_on_sc(ne, rcv, zeros):
    return _sc_calls()[1](ne, rcv, zeros)


def kernel(node_features, edge_features, senders, receivers, params):
    p = params

    pad = EP - E
    senders_p = jnp.concatenate(
        [senders, jnp.zeros((pad,), jnp.int32)]).reshape(EP // GCH, GCH)
    recv_gather_p = jnp.concatenate(
        [receivers, jnp.zeros((pad,), jnp.int32)]).reshape(EP // GCH, GCH)
    recv_scatter_p = jnp.concatenate(
        [receivers, jnp.full((pad,), N, jnp.int32)]).reshape(EP // GCH, GCH)
    ef_p = jnp.concatenate(
        [edge_features, jnp.zeros((pad, edge_features.shape[1]), _F32)])
    zeros_acc = jnp.zeros((NPAD, D), _F32)

    node_lat = _mlp3_ln(node_features, p['enc_node'], *p['enc_node_ln'],
                        blk=BLK_N)
    edge_lat = _mlp3_ln(ef_p, p['enc_edge'], *p['enc_edge_ln'], blk=BLK_E)

    hch = EPH // GCH
    snd_h = (senders_p[:hch], senders_p[hch:])
    rcvg_h = (recv_gather_p[:hch], recv_gather_p[hch:])
    rcvs_h = (recv_scatter_p[:hch], recv_scatter_p[hch:])
    el_h = [edge_lat[:EPH], edge_lat[EPH:]]

    for sp in p['steps']:
        (w1, b1), (w2, b2), (w3, b3) = sp['edge_mlp']
        ws, wr, we = w1[:D], w1[D:2 * D], w1[2 * D:]
        ps, pr = _proj2(node_lat, ws, wr)
        aggs = []
        g_h = [_gather_on_sc(ps, pr, snd_h[h], rcvg_h[h]) for h in range(2)]
        for h in range(2):
            gs, gr = g_h[h]
            new_e, el_h[h] = _edge_step(gs, gr, el_h[h], we, b1, w2, b2,
                                        w3, b3, *sp['edge_ln'])
            aggs.append(_scatter_on_sc(new_e, rcvs_h[h], zeros_acc))
        (wn1, bn1), (wn2, bn2), (wn3, bn3) = sp['node_mlp']
        node_lat = _node_step(node_lat,
                              aggs[0][:N], aggs[0][NPAD:NPAD + N],
                              aggs[1][:N], aggs[1][NPAD:NPAD + N],
                              wn1[:D], wn1[D:], bn1, wn2, bn2, wn3, bn3,
                              *sp['node_ln'])

    return _decoder(node_lat, p['dec'])
